# baseline (speedup 1.0000x reference)
import functools
import math

import jax
import jax.numpy as jnp
import numpy as np
from jax import lax
from jax.experimental import pallas as pl
from jax.experimental.pallas import tpu as pltpu
from jax.experimental.pallas import tpu_sc as plsc

MAX_RADIUS = 2.0
N_BASIS = 10
N = 10000
E = 320000
D = 128
HID = 100
NG = 16

NC = 2
NS = 16
NW = NC * NS
LANES = 16

HALF = D // 2
ROWS_PER_TILE = N // NS
EC_LEN = E // NW
CB = 128
NCHUNK = E // CB
CHUNK_ITERS = -(-NCHUNK // NS)

_f32 = jnp.float32


def _sc_mesh():
    return plsc.VectorSubcoreMesh(
        core_axis_name="c", subcore_axis_name="s", num_cores=NC, num_subcores=NS
    )


def _len2_body(px_hbm, py_hbm, pz_hbm, src_hbm, dst_hbm, out_hbm,
               px_v, py_v, pz_v, src_v, dst_v, out_v):
    c = lax.axis_index("c")
    s = lax.axis_index("s")
    wid = s * NC + c
    base = wid * EC_LEN
    pltpu.sync_copy(px_hbm, px_v)
    pltpu.sync_copy(py_hbm, py_v)
    pltpu.sync_copy(pz_hbm, pz_v)
    pltpu.sync_copy(src_hbm.at[pl.ds(base, EC_LEN)], src_v)
    pltpu.sync_copy(dst_hbm.at[pl.ds(base, EC_LEN)], dst_v)

    def body(i, carry):
        si = src_v[pl.ds(i * LANES, LANES)]
        di = dst_v[pl.ds(i * LANES, LANES)]
        ax = plsc.load_gather(px_v, [si]) - plsc.load_gather(px_v, [di])
        ay = plsc.load_gather(py_v, [si]) - plsc.load_gather(py_v, [di])
        az = plsc.load_gather(pz_v, [si]) - plsc.load_gather(pz_v, [di])
        out_v[pl.ds(i * LANES, LANES)] = ax * ax + ay * ay + az * az
        return carry

    lax.fori_loop(0, EC_LEN // LANES, body, 0)
    pltpu.sync_copy(out_v, out_hbm.at[pl.ds(base, EC_LEN)])


@functools.cache
def _len2_kernel():
    return pl.kernel(
        _len2_body,
        out_type=jax.ShapeDtypeStruct((E,), _f32),
        mesh=_sc_mesh(),
        compiler_params=pltpu.CompilerParams(needs_layout_passes=False, use_tc_tiling_on_sc=False),
        scratch_types=[
            pltpu.VMEM((N,), _f32),
            pltpu.VMEM((N,), _f32),
            pltpu.VMEM((N,), _f32),
            pltpu.VMEM((EC_LEN,), jnp.int32),
            pltpu.VMEM((EC_LEN,), jnp.int32),
            pltpu.VMEM((EC_LEN,), _f32),
        ],
    )


G = 2
TPC = -(-(-(-NCHUNK // NS)) // G) * G
NGROUP = TPC // G


def _edge_body(xl_hbm, wef_hbm, src2_hbm, dst2_hbm, zer_hbm, agg_hbm,
               xl_sh, agg_sh,
               srcA, dstA, srcB, dstB,
               wefA, wefB, gx0, gx1,
               sem_iA, sem_iB, sem_wA, sem_wB, gsem, ssem):
    c = lax.axis_index("c")
    s = lax.axis_index("s")
    rbase = s * ROWS_PER_TILE
    pltpu.sync_copy(xl_hbm.at[pl.ds(c * N + rbase, ROWS_PER_TILE)],
                    xl_sh.at[pl.ds(rbase, ROWS_PER_TILE)])
    pltpu.sync_copy(zer_hbm.at[pl.ds(rbase, ROWS_PER_TILE)],
                    agg_sh.at[pl.ds(rbase, ROWS_PER_TILE)])
    plsc.subcore_barrier()

    IDX = ((srcA, dstA, sem_iA), (srcB, dstB, sem_iB))
    WEFS = (wefA, wefB)
    SW = (sem_wA, sem_wB)
    GXS = (gx0, gx1)

    tbase = s * TPC
    tend = jnp.minimum(NCHUNK, tbase + TPC)

    def issue_group(g, p):
        gb = jnp.minimum(tbase + g * G, NCHUNK - G)
        srcb, dstb, semi = IDX[p]
        pltpu.async_copy(src2_hbm.at[pl.ds(gb, G)], srcb, semi)
        pltpu.async_copy(dst2_hbm.at[pl.ds(gb, G)], dstb, semi)
        pltpu.async_copy(wef_hbm.at[pl.ds(c * E + gb * CB, G * CB)],
                         WEFS[p], SW[p])

    def wait_idx(p):
        srcb, dstb, semi = IDX[p]
        pltpu.make_async_copy(src2_hbm.at[pl.ds(0, G)], srcb, semi).wait()
        pltpu.make_async_copy(dst2_hbm.at[pl.ds(0, G)], dstb, semi).wait()

    def wait_wef(p):
        pltpu.make_async_copy(wef_hbm.at[pl.ds(c * E, G * CB)],
                              WEFS[p], SW[p]).wait()

    def group_step(g, p):
        q = 1 - p
        srcb, dstb, _ = IDX[p]
        srcq, dstq, _ = IDX[q]
        gbase = tbase + g * G
        wait_idx(p)
        for j in range(G):
            kp = gbase - G + j

            @pl.when(jnp.logical_and(kp >= tbase, kp < tend))
            def _():
                pltpu.make_async_copy(GXS[j], agg_sh.at[dstq.at[j]],
                                      ssem).wait()
        for j in range(G):
            pltpu.async_copy(xl_sh.at[srcb.at[j]], GXS[j], gsem)
        issue_group(g + 1, q)
        wait_wef(p)
        for j in range(G):
            pltpu.make_async_copy(xl_sh.at[srcb.at[j]], GXS[j], gsem).wait()
        wefv = WEFS[p]
        for j in range(G):
            gxv = GXS[j]

            def mul(b4, carry):
                for bb in range(4):
                    b = b4 * 4 + bb
                    for m in range(HALF // LANES):
                        slc = pl.ds(m * LANES, LANES)
                        gxv[b, slc] = gxv[b, slc] * wefv[j * CB + b, slc]
                return carry

            lax.fori_loop(0, CB // 4, mul, 0)
            kj = gbase + j

            @pl.when(kj < tend)
            def _():
                pltpu.async_copy(gxv, agg_sh.at[dstb.at[j]], ssem, add=True)

    issue_group(0, 0)

    def two(t2, carry):
        group_step(t2 * 2, 0)
        group_step(t2 * 2 + 1, 1)
        return carry

    lax.fori_loop(0, NGROUP // 2, two, 0)
    if NGROUP % 2:
        group_step(NGROUP - 1, 0)

    srcq, dstq, _ = IDX[(NGROUP - 1) % 2]
    for j in range(G):
        kp = tbase + (NGROUP - 1) * G + j

        @pl.when(kp < tend)
        def _():
            pltpu.make_async_copy(GXS[j], agg_sh.at[dstq.at[j]], ssem).wait()
    wait_idx(NGROUP % 2)
    wait_wef(NGROUP % 2)

    plsc.subcore_barrier()
    pltpu.sync_copy(agg_sh.at[pl.ds(rbase, ROWS_PER_TILE)],
                    agg_hbm.at[pl.ds(c * N + rbase, ROWS_PER_TILE)])


@functools.cache
def _edge_kernel():
    return pl.kernel(
        _edge_body,
        out_type=jax.ShapeDtypeStruct((2 * N, HALF), _f32),
        mesh=_sc_mesh(),
        compiler_params=pltpu.CompilerParams(needs_layout_passes=False, use_tc_tiling_on_sc=False),
        scratch_types=[
            pltpu.VMEM_SHARED((N, HALF), _f32),
            pltpu.VMEM_SHARED((N, HALF), _f32),
            pltpu.VMEM((G, CB), jnp.int32),
            pltpu.VMEM((G, CB), jnp.int32),
            pltpu.VMEM((G, CB), jnp.int32),
            pltpu.VMEM((G, CB), jnp.int32),
            pltpu.VMEM((G * CB, HALF), _f32),
            pltpu.VMEM((G * CB, HALF), _f32),
            pltpu.VMEM((CB, HALF), _f32),
            pltpu.VMEM((CB, HALF), _f32),
        ] + [pltpu.SemaphoreType.DMA] * 6,
    )


EB = 2560
SQRT2 = math.sqrt(2.0)


def _wef_body(len2_ref, w1_ref, w2_ref, out_ref):
    l2 = len2_ref[0, 0, :]
    length = jnp.sqrt(l2 + 1e-12)
    centers = lax.broadcasted_iota(jnp.int32, (1, N_BASIS), 1).astype(_f32) * (
        MAX_RADIUS / (N_BASIS - 1))
    inv_sigma = (N_BASIS - 1) / MAX_RADIUS
    diff = (length[:, None] - centers) * inv_sigma
    emb = jnp.exp(-diff * diff)
    h1 = jnp.maximum(jnp.dot(emb, w1_ref[...],
                             preferred_element_type=_f32), 0.0) * SQRT2
    w = jnp.dot(h1, w2_ref[...], preferred_element_type=_f32) * (1.0 / math.sqrt(HID))
    u = 2.0 * (length * (1.0 / MAX_RADIUS) - 1.0)
    y = (1.0 - jnp.cos(jnp.pi * u)) * 0.5
    y = jnp.where(u > 0.0, 0.0, y)
    y = jnp.where(u < -1.0, 1.0, y)
    wef = w * y[:, None]
    out_ref[0] = wef[:, :HALF]
    out_ref[1] = wef[:, HALF:]


@functools.cache
def _wef_kernel():
    return pl.pallas_call(
        _wef_body,
        grid=(E // EB,),
        in_specs=[
            pl.BlockSpec((1, 1, EB), lambda i: (i, 0, 0)),
            pl.BlockSpec((N_BASIS, HID), lambda i: (0, 0)),
            pl.BlockSpec((HID, D), lambda i: (0, 0)),
        ],
        out_specs=pl.BlockSpec((2, EB, HALF), lambda i: (0, i, 0)),
        out_shape=jax.ShapeDtypeStruct((2, E, HALF), _f32),
    )


RB = 2000
INV_SQRT_D = 1.0 / math.sqrt(D)
AGG_SCALE = 0.5 / math.sqrt(32.0 * D)


def _node0_body(x_ref, wsi_ref, wl1_ref, si_ref, xl_ref):
    h = x_ref[...]
    si_ref[...] = jnp.dot(h, wsi_ref[...], preferred_element_type=_f32) * INV_SQRT_D
    xl = jnp.dot(h, wl1_ref[...], preferred_element_type=_f32) * INV_SQRT_D
    xl_ref[0] = xl[:, :HALF]
    xl_ref[1] = xl[:, HALF:]


def _node_mid_body(sip_ref, agg_ref, wl2_ref, wsi_ref, wl1_ref, si_ref, xl_ref):
    agg = jnp.concatenate([agg_ref[0], agg_ref[1]], axis=-1)
    h = sip_ref[...] + jnp.dot(agg, wl2_ref[...],
                               preferred_element_type=_f32) * AGG_SCALE
    h = jnp.maximum(h, 0.0)
    si_ref[...] = jnp.dot(h, wsi_ref[...], preferred_element_type=_f32) * INV_SQRT_D
    xl = jnp.dot(h, wl1_ref[...], preferred_element_type=_f32) * INV_SQRT_D
    xl_ref[0] = xl[:, :HALF]
    xl_ref[1] = xl[:, HALF:]


@functools.cache
def _node0_kernel():
    return pl.pallas_call(
        _node0_body,
        grid=(N // RB,),
        in_specs=[
            pl.BlockSpec((RB, D), lambda i: (i, 0)),
            pl.BlockSpec((D, D), lambda i: (0, 0)),
            pl.BlockSpec((D, D), lambda i: (0, 0)),
        ],
        out_specs=[
            pl.BlockSpec((RB, D), lambda i: (i, 0)),
            pl.BlockSpec((2, RB, HALF), lambda i: (0, i, 0)),
        ],
        out_shape=[
            jax.ShapeDtypeStruct((N, D), _f32),
            jax.ShapeDtypeStruct((2, N, HALF), _f32),
        ],
    )


@functools.cache
def _node_mid_kernel():
    return pl.pallas_call(
        _node_mid_body,
        grid=(N // RB,),
        in_specs=[
            pl.BlockSpec((RB, D), lambda i: (i, 0)),
            pl.BlockSpec((2, RB, HALF), lambda i: (0, i, 0)),
            pl.BlockSpec((D, D), lambda i: (0, 0)),
            pl.BlockSpec((D, D), lambda i: (0, 0)),
            pl.BlockSpec((D, D), lambda i: (0, 0)),
        ],
        out_specs=[
            pl.BlockSpec((RB, D), lambda i: (i, 0)),
            pl.BlockSpec((2, RB, HALF), lambda i: (0, i, 0)),
        ],
        out_shape=[
            jax.ShapeDtypeStruct((N, D), _f32),
            jax.ShapeDtypeStruct((2, N, HALF), _f32),
        ],
    )


INV_SQRT_NODES = 1.0 / math.sqrt(625.0)


def _final_body(sip_ref, agg_ref, wl2_ref, batch_ref, out_ref):
    agg = jnp.concatenate([agg_ref[0], agg_ref[1]], axis=-1)
    h = sip_ref[...] + jnp.dot(agg, wl2_ref[...],
                               preferred_element_type=_f32) * AGG_SCALE
    b = batch_ref[0, 0, :]
    gids = lax.broadcasted_iota(jnp.int32, (NG, RB), 0).astype(_f32)
    m = jnp.where(jnp.equal(b[None, :], gids), 1.0, 0.0)
    contrib = jnp.dot(m, h, preferred_element_type=_f32) * INV_SQRT_NODES

    @pl.when(pl.program_id(0) == 0)
    def _():
        out_ref[...] = jnp.zeros_like(out_ref)

    out_ref[...] += contrib


@functools.cache
def _final_kernel():
    return pl.pallas_call(
        _final_body,
        grid=(N // RB,),
        in_specs=[
            pl.BlockSpec((RB, D), lambda i: (i, 0)),
            pl.BlockSpec((2, RB, HALF), lambda i: (0, i, 0)),
            pl.BlockSpec((D, D), lambda i: (0, 0)),
            pl.BlockSpec((1, 1, RB), lambda i: (i, 0, 0)),
        ],
        out_specs=pl.BlockSpec((NG, D), lambda i: (0, 0)),
        out_shape=jax.ShapeDtypeStruct((NG, D), _f32),
    )


def kernel(pos, x, z, batch, edge_src, edge_dst,
           Wsi0, Wl1_0, Wfc1_0, Wfc2_0, Wl2_0,
           Wsi1, Wl1_1, Wfc1_1, Wfc2_1, Wl2_1,
           Wsi2, Wl1_2, Wfc1_2, Wfc2_2, Wl2_2):
    del z
    px = jnp.asarray(pos[:, 0], _f32)
    py = jnp.asarray(pos[:, 1], _f32)
    pz = jnp.asarray(pos[:, 2], _f32)
    src = edge_src.astype(jnp.int32)
    dst = edge_dst.astype(jnp.int32)

    len2 = _len2_kernel()(px, py, pz, src, dst)
    len2_3d = len2.reshape(E // EB, 1, EB)
    zer = jnp.zeros((N, HALF), _f32)
    batch3 = batch.astype(_f32).reshape(N // RB, 1, RB)

    wsis = [Wsi0[:, 0, :], Wsi1[:, 0, :], Wsi2[:, 0, :]]
    wl1s = [Wl1_0[:, 0, :], Wl1_1[:, 0, :], Wl1_2[:, 0, :]]
    wl2s = [Wl2_0[:, 0, :], Wl2_1[:, 0, :], Wl2_2[:, 0, :]]
    wfc1s = [Wfc1_0, Wfc1_1, Wfc1_2]
    wfc2s = [Wfc2_0, Wfc2_1, Wfc2_2]

    si, xl2 = _node0_kernel()(x, wsis[0], wl1s[0])
    for l in range(3):
        wef2 = _wef_kernel()(len2_3d, wfc1s[l], wfc2s[l])
        agg_flat = _edge_kernel()(
            xl2.reshape(2 * N, HALF), wef2.reshape(2 * E, HALF),
            src.reshape(NCHUNK, CB), dst.reshape(NCHUNK, CB), zer)
        agg2 = agg_flat.reshape(2, N, HALF)
        if l < 2:
            si, xl2 = _node_mid_kernel()(si, agg2, wl2s[l],
                                         wsis[l + 1], wl1s[l + 1])
    return _final_kernel()(si, agg2, wl2s[2], batch3)


# speedup vs baseline: 1.0629x; 1.0629x over previous
import functools
import math

import jax
import jax.numpy as jnp
import numpy as np
from jax import lax
from jax.experimental import pallas as pl
from jax.experimental.pallas import tpu as pltpu
from jax.experimental.pallas import tpu_sc as plsc

MAX_RADIUS = 2.0
N_BASIS = 10
N = 10000
E = 320000
D = 128
HID = 100
NG = 16

NC = 2
NS = 16
NW = NC * NS
LANES = 16

HALF = D // 2
ROWS_PER_TILE = N // NS
EC_LEN = E // NW
CB = 128
NCHUNK = E // CB
CHUNK_ITERS = -(-NCHUNK // NS)

_f32 = jnp.float32


def _sc_mesh():
    return plsc.VectorSubcoreMesh(
        core_axis_name="c", subcore_axis_name="s", num_cores=NC, num_subcores=NS
    )


def _len2_body(px_hbm, py_hbm, pz_hbm, src_hbm, dst_hbm, out_hbm,
               px_v, py_v, pz_v, src_v, dst_v, out_v):
    c = lax.axis_index("c")
    s = lax.axis_index("s")
    wid = s * NC + c
    base = wid * EC_LEN
    pltpu.sync_copy(px_hbm, px_v)
    pltpu.sync_copy(py_hbm, py_v)
    pltpu.sync_copy(pz_hbm, pz_v)
    pltpu.sync_copy(src_hbm.at[pl.ds(base, EC_LEN)], src_v)
    pltpu.sync_copy(dst_hbm.at[pl.ds(base, EC_LEN)], dst_v)

    def body(i, carry):
        si = src_v[pl.ds(i * LANES, LANES)]
        di = dst_v[pl.ds(i * LANES, LANES)]
        ax = plsc.load_gather(px_v, [si]) - plsc.load_gather(px_v, [di])
        ay = plsc.load_gather(py_v, [si]) - plsc.load_gather(py_v, [di])
        az = plsc.load_gather(pz_v, [si]) - plsc.load_gather(pz_v, [di])
        out_v[pl.ds(i * LANES, LANES)] = ax * ax + ay * ay + az * az
        return carry

    lax.fori_loop(0, EC_LEN // LANES, body, 0)
    pltpu.sync_copy(out_v, out_hbm.at[pl.ds(base, EC_LEN)])


@functools.cache
def _len2_kernel():
    return pl.kernel(
        _len2_body,
        out_type=jax.ShapeDtypeStruct((E,), _f32),
        mesh=_sc_mesh(),
        compiler_params=pltpu.CompilerParams(needs_layout_passes=False, use_tc_tiling_on_sc=False),
        scratch_types=[
            pltpu.VMEM((N,), _f32),
            pltpu.VMEM((N,), _f32),
            pltpu.VMEM((N,), _f32),
            pltpu.VMEM((EC_LEN,), jnp.int32),
            pltpu.VMEM((EC_LEN,), jnp.int32),
            pltpu.VMEM((EC_LEN,), _f32),
        ],
    )


G = 2
TPC = -(-(-(-NCHUNK // NS)) // G) * G
NGROUP = TPC // G


def _edge_body(xl_hbm, wef_hbm, src2_hbm, dst2_hbm, zer_hbm, agg_hbm,
               xl_sh, agg_sh,
               srcA, dstA, srcB, dstB,
               wefA, wefB, gx0, gx1,
               sem_iA, sem_iB, sem_wA, sem_wB, gsem, ssem):
    c = lax.axis_index("c")
    s = lax.axis_index("s")
    rbase = s * ROWS_PER_TILE
    pltpu.sync_copy(xl_hbm.at[pl.ds(c * N + rbase, ROWS_PER_TILE)],
                    xl_sh.at[pl.ds(rbase, ROWS_PER_TILE)])
    pltpu.sync_copy(zer_hbm.at[pl.ds(rbase, ROWS_PER_TILE)],
                    agg_sh.at[pl.ds(rbase, ROWS_PER_TILE)])
    plsc.subcore_barrier()

    IDX = ((srcA, dstA, sem_iA), (srcB, dstB, sem_iB))
    WEFS = (wefA, wefB)
    SW = (sem_wA, sem_wB)
    GXS = (gx0, gx1)

    tbase = s * TPC
    tend = jnp.minimum(NCHUNK, tbase + TPC)

    def issue_group(g, p):
        gb = jnp.minimum(tbase + g * G, NCHUNK - G)
        srcb, dstb, semi = IDX[p]
        pltpu.async_copy(src2_hbm.at[pl.ds(gb, G)], srcb, semi)
        pltpu.async_copy(dst2_hbm.at[pl.ds(gb, G)], dstb, semi)
        pltpu.async_copy(wef_hbm.at[pl.ds(c * E + gb * CB, G * CB)],
                         WEFS[p], SW[p])

    def wait_idx(p):
        srcb, dstb, semi = IDX[p]
        pltpu.make_async_copy(src2_hbm.at[pl.ds(0, G)], srcb, semi).wait()
        pltpu.make_async_copy(dst2_hbm.at[pl.ds(0, G)], dstb, semi).wait()

    def wait_wef(p):
        pltpu.make_async_copy(wef_hbm.at[pl.ds(c * E, G * CB)],
                              WEFS[p], SW[p]).wait()

    def group_step(g, p):
        q = 1 - p
        srcb, dstb, _ = IDX[p]
        srcq, dstq, _ = IDX[q]
        gbase = tbase + g * G
        wait_idx(p)
        for j in range(G):
            kp = gbase - G + j

            @pl.when(jnp.logical_and(kp >= tbase, kp < tend))
            def _():
                pltpu.make_async_copy(GXS[j], agg_sh.at[dstq.at[j]],
                                      ssem).wait()
        for j in range(G):
            pltpu.async_copy(xl_sh.at[srcb.at[j]], GXS[j], gsem)
        issue_group(g + 1, q)
        wait_wef(p)
        for j in range(G):
            pltpu.make_async_copy(xl_sh.at[srcb.at[j]], GXS[j], gsem).wait()
        wefv = WEFS[p]
        for j in range(G):
            gxv = GXS[j]

            def mul(b4, carry):
                for bb in range(4):
                    b = b4 * 4 + bb
                    for m in range(HALF // LANES):
                        slc = pl.ds(m * LANES, LANES)
                        gxv[b, slc] = gxv[b, slc] * wefv[j * CB + b, slc]
                return carry

            lax.fori_loop(0, CB // 4, mul, 0)
            kj = gbase + j

            @pl.when(kj < tend)
            def _():
                pltpu.async_copy(gxv, agg_sh.at[dstb.at[j]], ssem, add=True)

    issue_group(0, 0)

    def two(t2, carry):
        group_step(t2 * 2, 0)
        group_step(t2 * 2 + 1, 1)
        return carry

    lax.fori_loop(0, NGROUP // 2, two, 0)
    if NGROUP % 2:
        group_step(NGROUP - 1, 0)

    srcq, dstq, _ = IDX[(NGROUP - 1) % 2]
    for j in range(G):
        kp = tbase + (NGROUP - 1) * G + j

        @pl.when(kp < tend)
        def _():
            pltpu.make_async_copy(GXS[j], agg_sh.at[dstq.at[j]], ssem).wait()
    wait_idx(NGROUP % 2)
    wait_wef(NGROUP % 2)

    plsc.subcore_barrier()
    pltpu.sync_copy(agg_sh.at[pl.ds(rbase, ROWS_PER_TILE)],
                    agg_hbm.at[pl.ds(c * N + rbase, ROWS_PER_TILE)])


@functools.cache
def _edge_kernel():
    return pl.kernel(
        _edge_body,
        out_type=jax.ShapeDtypeStruct((2 * N, HALF), _f32),
        mesh=_sc_mesh(),
        compiler_params=pltpu.CompilerParams(needs_layout_passes=False, use_tc_tiling_on_sc=False),
        scratch_types=[
            pltpu.VMEM_SHARED((N, HALF), _f32),
            pltpu.VMEM_SHARED((N, HALF), _f32),
            pltpu.VMEM((G, CB), jnp.int32),
            pltpu.VMEM((G, CB), jnp.int32),
            pltpu.VMEM((G, CB), jnp.int32),
            pltpu.VMEM((G, CB), jnp.int32),
            pltpu.VMEM((G * CB, HALF), _f32),
            pltpu.VMEM((G * CB, HALF), _f32),
            pltpu.VMEM((CB, HALF), _f32),
            pltpu.VMEM((CB, HALF), _f32),
        ] + [pltpu.SemaphoreType.DMA] * 6,
    )


EB = 2560
SQRT2 = math.sqrt(2.0)


def _wef_body(len2_ref, w1a_ref, w2a_ref, w1b_ref, w2b_ref, w1c_ref, w2c_ref,
              outa_ref, outb_ref, outc_ref):
    l2 = len2_ref[0, 0, :]
    length = jnp.sqrt(l2 + 1e-12)
    centers = lax.broadcasted_iota(jnp.int32, (1, N_BASIS), 1).astype(_f32) * (
        MAX_RADIUS / (N_BASIS - 1))
    inv_sigma = (N_BASIS - 1) / MAX_RADIUS
    diff = (length[:, None] - centers) * inv_sigma
    emb = jnp.exp(-diff * diff)
    u = 2.0 * (length * (1.0 / MAX_RADIUS) - 1.0)
    y = (1.0 - jnp.cos(jnp.pi * u)) * 0.5
    y = jnp.where(u > 0.0, 0.0, y)
    y = jnp.where(u < -1.0, 1.0, y)
    for w1_ref, w2_ref, out_ref in ((w1a_ref, w2a_ref, outa_ref),
                                    (w1b_ref, w2b_ref, outb_ref),
                                    (w1c_ref, w2c_ref, outc_ref)):
        h1 = jnp.maximum(jnp.dot(emb, w1_ref[...],
                                 preferred_element_type=_f32), 0.0) * SQRT2
        w = jnp.dot(h1, w2_ref[...],
                    preferred_element_type=_f32) * (1.0 / math.sqrt(HID))
        wef = w * y[:, None]
        out_ref[0] = wef[:, :HALF]
        out_ref[1] = wef[:, HALF:]


@functools.cache
def _wef_kernel():
    wspec = [
        pl.BlockSpec((N_BASIS, HID), lambda i: (0, 0)),
        pl.BlockSpec((HID, D), lambda i: (0, 0)),
    ]
    ospec = pl.BlockSpec((2, EB, HALF), lambda i: (0, i, 0))
    oshape = jax.ShapeDtypeStruct((2, E, HALF), _f32)
    return pl.pallas_call(
        _wef_body,
        grid=(E // EB,),
        in_specs=[pl.BlockSpec((1, 1, EB), lambda i: (i, 0, 0))] + wspec * 3,
        out_specs=[ospec] * 3,
        out_shape=[oshape] * 3,
    )


RB = 2000
INV_SQRT_D = 1.0 / math.sqrt(D)
AGG_SCALE = 0.5 / math.sqrt(32.0 * D)


def _node0_body(x_ref, wsi_ref, wl1_ref, si_ref, xl_ref):
    h = x_ref[...]
    si_ref[...] = jnp.dot(h, wsi_ref[...], preferred_element_type=_f32) * INV_SQRT_D
    xl = jnp.dot(h, wl1_ref[...], preferred_element_type=_f32) * INV_SQRT_D
    xl_ref[0] = xl[:, :HALF]
    xl_ref[1] = xl[:, HALF:]


def _node_mid_body(sip_ref, agg_ref, wl2_ref, wsi_ref, wl1_ref, si_ref, xl_ref):
    agg = jnp.concatenate([agg_ref[0], agg_ref[1]], axis=-1)
    h = sip_ref[...] + jnp.dot(agg, wl2_ref[...],
                               preferred_element_type=_f32) * AGG_SCALE
    h = jnp.maximum(h, 0.0)
    si_ref[...] = jnp.dot(h, wsi_ref[...], preferred_element_type=_f32) * INV_SQRT_D
    xl = jnp.dot(h, wl1_ref[...], preferred_element_type=_f32) * INV_SQRT_D
    xl_ref[0] = xl[:, :HALF]
    xl_ref[1] = xl[:, HALF:]


@functools.cache
def _node0_kernel():
    return pl.pallas_call(
        _node0_body,
        grid=(N // RB,),
        in_specs=[
            pl.BlockSpec((RB, D), lambda i: (i, 0)),
            pl.BlockSpec((D, D), lambda i: (0, 0)),
            pl.BlockSpec((D, D), lambda i: (0, 0)),
        ],
        out_specs=[
            pl.BlockSpec((RB, D), lambda i: (i, 0)),
            pl.BlockSpec((2, RB, HALF), lambda i: (0, i, 0)),
        ],
        out_shape=[
            jax.ShapeDtypeStruct((N, D), _f32),
            jax.ShapeDtypeStruct((2, N, HALF), _f32),
        ],
    )


@functools.cache
def _node_mid_kernel():
    return pl.pallas_call(
        _node_mid_body,
        grid=(N // RB,),
        in_specs=[
            pl.BlockSpec((RB, D), lambda i: (i, 0)),
            pl.BlockSpec((2, RB, HALF), lambda i: (0, i, 0)),
            pl.BlockSpec((D, D), lambda i: (0, 0)),
            pl.BlockSpec((D, D), lambda i: (0, 0)),
            pl.BlockSpec((D, D), lambda i: (0, 0)),
        ],
        out_specs=[
            pl.BlockSpec((RB, D), lambda i: (i, 0)),
            pl.BlockSpec((2, RB, HALF), lambda i: (0, i, 0)),
        ],
        out_shape=[
            jax.ShapeDtypeStruct((N, D), _f32),
            jax.ShapeDtypeStruct((2, N, HALF), _f32),
        ],
    )


INV_SQRT_NODES = 1.0 / math.sqrt(625.0)


def _final_body(sip_ref, agg_ref, wl2_ref, batch_ref, out_ref):
    agg = jnp.concatenate([agg_ref[0], agg_ref[1]], axis=-1)
    h = sip_ref[...] + jnp.dot(agg, wl2_ref[...],
                               preferred_element_type=_f32) * AGG_SCALE
    b = batch_ref[0, 0, :]
    gids = lax.broadcasted_iota(jnp.int32, (NG, RB), 0).astype(_f32)
    m = jnp.where(jnp.equal(b[None, :], gids), 1.0, 0.0)
    contrib = jnp.dot(m, h, preferred_element_type=_f32) * INV_SQRT_NODES

    @pl.when(pl.program_id(0) == 0)
    def _():
        out_ref[...] = jnp.zeros_like(out_ref)

    out_ref[...] += contrib


@functools.cache
def _final_kernel():
    return pl.pallas_call(
        _final_body,
        grid=(N // RB,),
        in_specs=[
            pl.BlockSpec((RB, D), lambda i: (i, 0)),
            pl.BlockSpec((2, RB, HALF), lambda i: (0, i, 0)),
            pl.BlockSpec((D, D), lambda i: (0, 0)),
            pl.BlockSpec((1, 1, RB), lambda i: (i, 0, 0)),
        ],
        out_specs=pl.BlockSpec((NG, D), lambda i: (0, 0)),
        out_shape=jax.ShapeDtypeStruct((NG, D), _f32),
    )


def kernel(pos, x, z, batch, edge_src, edge_dst,
           Wsi0, Wl1_0, Wfc1_0, Wfc2_0, Wl2_0,
           Wsi1, Wl1_1, Wfc1_1, Wfc2_1, Wl2_1,
           Wsi2, Wl1_2, Wfc1_2, Wfc2_2, Wl2_2):
    del z
    px = jnp.asarray(pos[:, 0], _f32)
    py = jnp.asarray(pos[:, 1], _f32)
    pz = jnp.asarray(pos[:, 2], _f32)
    src = edge_src.astype(jnp.int32)
    dst = edge_dst.astype(jnp.int32)

    len2 = _len2_kernel()(px, py, pz, src, dst)
    len2_3d = len2.reshape(E // EB, 1, EB)
    zer = jnp.zeros((N, HALF), _f32)
    batch3 = batch.astype(_f32).reshape(N // RB, 1, RB)

    wsis = [Wsi0[:, 0, :], Wsi1[:, 0, :], Wsi2[:, 0, :]]
    wl1s = [Wl1_0[:, 0, :], Wl1_1[:, 0, :], Wl1_2[:, 0, :]]
    wl2s = [Wl2_0[:, 0, :], Wl2_1[:, 0, :], Wl2_2[:, 0, :]]
    wfc1s = [Wfc1_0, Wfc1_1, Wfc1_2]
    wfc2s = [Wfc2_0, Wfc2_1, Wfc2_2]

    si, xl2 = _node0_kernel()(x, wsis[0], wl1s[0])
    wef_all = _wef_kernel()(len2_3d, wfc1s[0], wfc2s[0], wfc1s[1], wfc2s[1],
                            wfc1s[2], wfc2s[2])
    for l in range(3):
        agg_flat = _edge_kernel()(
            xl2.reshape(2 * N, HALF), wef_all[l].reshape(2 * E, HALF),
            src.reshape(NCHUNK, CB), dst.reshape(NCHUNK, CB), zer)
        agg2 = agg_flat.reshape(2, N, HALF)
        if l < 2:
            si, xl2 = _node_mid_kernel()(si, agg2, wl2s[l],
                                         wsis[l + 1], wl1s[l + 1])
    return _final_kernel()(si, agg2, wl2s[2], batch3)


# speedup vs baseline: 1.0636x; 1.0007x over previous
import functools
import math

import jax
import jax.numpy as jnp
import numpy as np
from jax import lax
from jax.experimental import pallas as pl
from jax.experimental.pallas import tpu as pltpu
from jax.experimental.pallas import tpu_sc as plsc

MAX_RADIUS = 2.0
N_BASIS = 10
N = 10000
E = 320000
D = 128
HID = 100
NG = 16

NC = 2
NS = 16
NW = NC * NS
LANES = 16

HALF = D // 2
ROWS_PER_TILE = N // NS
EC_LEN = E // NW
CB = 128
NCHUNK = E // CB
CHUNK_ITERS = -(-NCHUNK // NS)

_f32 = jnp.float32


def _sc_mesh():
    return plsc.VectorSubcoreMesh(
        core_axis_name="c", subcore_axis_name="s", num_cores=NC, num_subcores=NS
    )


def _len2_body(px_hbm, py_hbm, pz_hbm, src_hbm, dst_hbm, out_hbm,
               px_v, py_v, pz_v, src_v, dst_v, out_v):
    c = lax.axis_index("c")
    s = lax.axis_index("s")
    wid = s * NC + c
    base = wid * EC_LEN
    pltpu.sync_copy(px_hbm, px_v)
    pltpu.sync_copy(py_hbm, py_v)
    pltpu.sync_copy(pz_hbm, pz_v)
    pltpu.sync_copy(src_hbm.at[pl.ds(base, EC_LEN)], src_v)
    pltpu.sync_copy(dst_hbm.at[pl.ds(base, EC_LEN)], dst_v)

    def body(i, carry):
        si = src_v[pl.ds(i * LANES, LANES)]
        di = dst_v[pl.ds(i * LANES, LANES)]
        ax = plsc.load_gather(px_v, [si]) - plsc.load_gather(px_v, [di])
        ay = plsc.load_gather(py_v, [si]) - plsc.load_gather(py_v, [di])
        az = plsc.load_gather(pz_v, [si]) - plsc.load_gather(pz_v, [di])
        out_v[pl.ds(i * LANES, LANES)] = ax * ax + ay * ay + az * az
        return carry

    lax.fori_loop(0, EC_LEN // LANES, body, 0)
    pltpu.sync_copy(out_v, out_hbm.at[pl.ds(base, EC_LEN)])


@functools.cache
def _len2_kernel():
    return pl.kernel(
        _len2_body,
        out_type=jax.ShapeDtypeStruct((E,), _f32),
        mesh=_sc_mesh(),
        compiler_params=pltpu.CompilerParams(needs_layout_passes=False, use_tc_tiling_on_sc=False),
        scratch_types=[
            pltpu.VMEM((N,), _f32),
            pltpu.VMEM((N,), _f32),
            pltpu.VMEM((N,), _f32),
            pltpu.VMEM((EC_LEN,), jnp.int32),
            pltpu.VMEM((EC_LEN,), jnp.int32),
            pltpu.VMEM((EC_LEN,), _f32),
        ],
    )


G = 2
TPC = -(-(-(-NCHUNK // NS)) // G) * G
NGROUP = TPC // G


def _edge_body(xl_hbm, wef_hbm, src2_hbm, dst2_hbm, zer_hbm, agg_hbm,
               xl_sh, agg_sh,
               srcA, dstA, srcB, dstB,
               wefA, wefB, gx0, gx1,
               sem_iA, sem_iB, sem_wA, sem_wB, gsem, ssem):
    c = lax.axis_index("c")
    s = lax.axis_index("s")
    rbase = s * ROWS_PER_TILE
    pltpu.sync_copy(xl_hbm.at[pl.ds(c * N + rbase, ROWS_PER_TILE)],
                    xl_sh.at[pl.ds(rbase, ROWS_PER_TILE)])
    pltpu.sync_copy(zer_hbm.at[pl.ds(rbase, ROWS_PER_TILE)],
                    agg_sh.at[pl.ds(rbase, ROWS_PER_TILE)])
    plsc.subcore_barrier()

    IDX = ((srcA, dstA, sem_iA), (srcB, dstB, sem_iB))
    WEFS = (wefA, wefB)
    SW = (sem_wA, sem_wB)
    GXS = (gx0, gx1)

    tbase = s * TPC
    tend = jnp.minimum(NCHUNK, tbase + TPC)

    def issue_group(g, p):
        gb = jnp.minimum(tbase + g * G, NCHUNK - G)
        srcb, dstb, semi = IDX[p]
        pltpu.async_copy(src2_hbm.at[pl.ds(gb, G)], srcb, semi)
        pltpu.async_copy(dst2_hbm.at[pl.ds(gb, G)], dstb, semi)
        pltpu.async_copy(wef_hbm.at[pl.ds(c * E + gb * CB, G * CB)],
                         WEFS[p], SW[p])

    def wait_idx(p):
        srcb, dstb, semi = IDX[p]
        pltpu.make_async_copy(src2_hbm.at[pl.ds(0, G)], srcb, semi).wait()
        pltpu.make_async_copy(dst2_hbm.at[pl.ds(0, G)], dstb, semi).wait()

    def wait_wef(p):
        pltpu.make_async_copy(wef_hbm.at[pl.ds(c * E, G * CB)],
                              WEFS[p], SW[p]).wait()

    def group_step(g, p):
        q = 1 - p
        srcb, dstb, _ = IDX[p]
        srcq, dstq, _ = IDX[q]
        gbase = tbase + g * G
        wait_idx(p)
        for j in range(G):
            kp = gbase - G + j

            @pl.when(jnp.logical_and(kp >= tbase, kp < tend))
            def _():
                pltpu.make_async_copy(GXS[j], agg_sh.at[dstq.at[j]],
                                      ssem).wait()
        for j in range(G):
            pltpu.async_copy(xl_sh.at[srcb.at[j]], GXS[j], gsem)
        issue_group(g + 1, q)
        wait_wef(p)
        for j in range(G):
            pltpu.make_async_copy(xl_sh.at[srcb.at[j]], GXS[j], gsem).wait()
        wefv = WEFS[p]
        for j in range(G):
            gxv = GXS[j]

            def mul(b4, carry):
                for bb in range(4):
                    b = b4 * 4 + bb
                    for m in range(HALF // LANES):
                        slc = pl.ds(m * LANES, LANES)
                        gxv[b, slc] = gxv[b, slc] * wefv[j * CB + b, slc]
                return carry

            lax.fori_loop(0, CB // 4, mul, 0)
            kj = gbase + j

            @pl.when(kj < tend)
            def _():
                pltpu.async_copy(gxv, agg_sh.at[dstb.at[j]], ssem, add=True)

    issue_group(0, 0)

    def two(t2, carry):
        group_step(t2 * 2, 0)
        group_step(t2 * 2 + 1, 1)
        return carry

    lax.fori_loop(0, NGROUP // 2, two, 0)
    if NGROUP % 2:
        group_step(NGROUP - 1, 0)

    srcq, dstq, _ = IDX[(NGROUP - 1) % 2]
    for j in range(G):
        kp = tbase + (NGROUP - 1) * G + j

        @pl.when(kp < tend)
        def _():
            pltpu.make_async_copy(GXS[j], agg_sh.at[dstq.at[j]], ssem).wait()
    wait_idx(NGROUP % 2)
    wait_wef(NGROUP % 2)

    plsc.subcore_barrier()
    pltpu.sync_copy(agg_sh.at[pl.ds(rbase, ROWS_PER_TILE)],
                    agg_hbm.at[pl.ds(c * N + rbase, ROWS_PER_TILE)])


@functools.cache
def _edge_kernel():
    return pl.kernel(
        _edge_body,
        out_type=jax.ShapeDtypeStruct((2 * N, HALF), _f32),
        mesh=_sc_mesh(),
        compiler_params=pltpu.CompilerParams(needs_layout_passes=False, use_tc_tiling_on_sc=False),
        scratch_types=[
            pltpu.VMEM_SHARED((N, HALF), _f32),
            pltpu.VMEM_SHARED((N, HALF), _f32),
            pltpu.VMEM((G, CB), jnp.int32),
            pltpu.VMEM((G, CB), jnp.int32),
            pltpu.VMEM((G, CB), jnp.int32),
            pltpu.VMEM((G, CB), jnp.int32),
            pltpu.VMEM((G * CB, HALF), _f32),
            pltpu.VMEM((G * CB, HALF), _f32),
            pltpu.VMEM((CB, HALF), _f32),
            pltpu.VMEM((CB, HALF), _f32),
        ] + [pltpu.SemaphoreType.DMA] * 6,
    )


EB = 2560
SQRT2 = math.sqrt(2.0)


def _wef_body(len2_ref, w1a_ref, w2a_ref, w1b_ref, w2b_ref, w1c_ref, w2c_ref,
              outa_ref, outb_ref, outc_ref):
    l2 = len2_ref[0, 0, :]
    length = jnp.sqrt(l2 + 1e-12)
    centers = lax.broadcasted_iota(jnp.int32, (1, N_BASIS), 1).astype(_f32) * (
        MAX_RADIUS / (N_BASIS - 1))
    inv_sigma = (N_BASIS - 1) / MAX_RADIUS
    diff = (length[:, None] - centers) * inv_sigma
    emb = jnp.exp(-diff * diff)
    u = 2.0 * (length * (1.0 / MAX_RADIUS) - 1.0)
    y = (1.0 - jnp.cos(jnp.pi * u)) * 0.5
    y = jnp.where(u > 0.0, 0.0, y)
    y = jnp.where(u < -1.0, 1.0, y)
    for w1_ref, w2_ref, out_ref in ((w1a_ref, w2a_ref, outa_ref),
                                    (w1b_ref, w2b_ref, outb_ref),
                                    (w1c_ref, w2c_ref, outc_ref)):
        h1 = jnp.maximum(jnp.dot(emb, w1_ref[...],
                                 preferred_element_type=_f32), 0.0) * SQRT2
        w = jnp.dot(h1.astype(jnp.bfloat16), w2_ref[...].astype(jnp.bfloat16),
                    preferred_element_type=_f32) * (1.0 / math.sqrt(HID))
        wef = w * y[:, None]
        out_ref[0] = wef[:, :HALF]
        out_ref[1] = wef[:, HALF:]


@functools.cache
def _wef_kernel():
    wspec = [
        pl.BlockSpec((N_BASIS, HID), lambda i: (0, 0)),
        pl.BlockSpec((HID, D), lambda i: (0, 0)),
    ]
    ospec = pl.BlockSpec((2, EB, HALF), lambda i: (0, i, 0))
    oshape = jax.ShapeDtypeStruct((2, E, HALF), _f32)
    return pl.pallas_call(
        _wef_body,
        grid=(E // EB,),
        in_specs=[pl.BlockSpec((1, 1, EB), lambda i: (i, 0, 0))] + wspec * 3,
        out_specs=[ospec] * 3,
        out_shape=[oshape] * 3,
    )


RB = 2000
INV_SQRT_D = 1.0 / math.sqrt(D)
AGG_SCALE = 0.5 / math.sqrt(32.0 * D)


def _node0_body(x_ref, wsi_ref, wl1_ref, si_ref, xl_ref):
    h = x_ref[...]
    si_ref[...] = jnp.dot(h, wsi_ref[...], preferred_element_type=_f32) * INV_SQRT_D
    xl = jnp.dot(h, wl1_ref[...], preferred_element_type=_f32) * INV_SQRT_D
    xl_ref[0] = xl[:, :HALF]
    xl_ref[1] = xl[:, HALF:]


def _node_mid_body(sip_ref, agg_ref, wl2_ref, wsi_ref, wl1_ref, si_ref, xl_ref):
    agg = jnp.concatenate([agg_ref[0], agg_ref[1]], axis=-1)
    h = sip_ref[...] + jnp.dot(agg, wl2_ref[...],
                               preferred_element_type=_f32) * AGG_SCALE
    h = jnp.maximum(h, 0.0)
    si_ref[...] = jnp.dot(h, wsi_ref[...], preferred_element_type=_f32) * INV_SQRT_D
    xl = jnp.dot(h, wl1_ref[...], preferred_element_type=_f32) * INV_SQRT_D
    xl_ref[0] = xl[:, :HALF]
    xl_ref[1] = xl[:, HALF:]


@functools.cache
def _node0_kernel():
    return pl.pallas_call(
        _node0_body,
        grid=(N // RB,),
        in_specs=[
            pl.BlockSpec((RB, D), lambda i: (i, 0)),
            pl.BlockSpec((D, D), lambda i: (0, 0)),
            pl.BlockSpec((D, D), lambda i: (0, 0)),
        ],
        out_specs=[
            pl.BlockSpec((RB, D), lambda i: (i, 0)),
            pl.BlockSpec((2, RB, HALF), lambda i: (0, i, 0)),
        ],
        out_shape=[
            jax.ShapeDtypeStruct((N, D), _f32),
            jax.ShapeDtypeStruct((2, N, HALF), _f32),
        ],
    )


@functools.cache
def _node_mid_kernel():
    return pl.pallas_call(
        _node_mid_body,
        grid=(N // RB,),
        in_specs=[
            pl.BlockSpec((RB, D), lambda i: (i, 0)),
            pl.BlockSpec((2, RB, HALF), lambda i: (0, i, 0)),
            pl.BlockSpec((D, D), lambda i: (0, 0)),
            pl.BlockSpec((D, D), lambda i: (0, 0)),
            pl.BlockSpec((D, D), lambda i: (0, 0)),
        ],
        out_specs=[
            pl.BlockSpec((RB, D), lambda i: (i, 0)),
            pl.BlockSpec((2, RB, HALF), lambda i: (0, i, 0)),
        ],
        out_shape=[
            jax.ShapeDtypeStruct((N, D), _f32),
            jax.ShapeDtypeStruct((2, N, HALF), _f32),
        ],
    )


INV_SQRT_NODES = 1.0 / math.sqrt(625.0)


def _final_body(sip_ref, agg_ref, wl2_ref, batch_ref, out_ref):
    agg = jnp.concatenate([agg_ref[0], agg_ref[1]], axis=-1)
    h = sip_ref[...] + jnp.dot(agg, wl2_ref[...],
                               preferred_element_type=_f32) * AGG_SCALE
    b = batch_ref[0, 0, :]
    gids = lax.broadcasted_iota(jnp.int32, (NG, RB), 0).astype(_f32)
    m = jnp.where(jnp.equal(b[None, :], gids), 1.0, 0.0)
    contrib = jnp.dot(m, h, preferred_element_type=_f32) * INV_SQRT_NODES

    @pl.when(pl.program_id(0) == 0)
    def _():
        out_ref[...] = jnp.zeros_like(out_ref)

    out_ref[...] += contrib


@functools.cache
def _final_kernel():
    return pl.pallas_call(
        _final_body,
        grid=(N // RB,),
        in_specs=[
            pl.BlockSpec((RB, D), lambda i: (i, 0)),
            pl.BlockSpec((2, RB, HALF), lambda i: (0, i, 0)),
            pl.BlockSpec((D, D), lambda i: (0, 0)),
            pl.BlockSpec((1, 1, RB), lambda i: (i, 0, 0)),
        ],
        out_specs=pl.BlockSpec((NG, D), lambda i: (0, 0)),
        out_shape=jax.ShapeDtypeStruct((NG, D), _f32),
    )


def kernel(pos, x, z, batch, edge_src, edge_dst,
           Wsi0, Wl1_0, Wfc1_0, Wfc2_0, Wl2_0,
           Wsi1, Wl1_1, Wfc1_1, Wfc2_1, Wl2_1,
           Wsi2, Wl1_2, Wfc1_2, Wfc2_2, Wl2_2):
    del z
    px = jnp.asarray(pos[:, 0], _f32)
    py = jnp.asarray(pos[:, 1], _f32)
    pz = jnp.asarray(pos[:, 2], _f32)
    src = edge_src.astype(jnp.int32)
    dst = edge_dst.astype(jnp.int32)

    len2 = _len2_kernel()(px, py, pz, src, dst)
    len2_3d = len2.reshape(E // EB, 1, EB)
    zer = jnp.zeros((N, HALF), _f32)
    batch3 = batch.astype(_f32).reshape(N // RB, 1, RB)

    wsis = [Wsi0[:, 0, :], Wsi1[:, 0, :], Wsi2[:, 0, :]]
    wl1s = [Wl1_0[:, 0, :], Wl1_1[:, 0, :], Wl1_2[:, 0, :]]
    wl2s = [Wl2_0[:, 0, :], Wl2_1[:, 0, :], Wl2_2[:, 0, :]]
    wfc1s = [Wfc1_0, Wfc1_1, Wfc1_2]
    wfc2s = [Wfc2_0, Wfc2_1, Wfc2_2]

    si, xl2 = _node0_kernel()(x, wsis[0], wl1s[0])
    wef_all = _wef_kernel()(len2_3d, wfc1s[0], wfc2s[0], wfc1s[1], wfc2s[1],
                            wfc1s[2], wfc2s[2])
    for l in range(3):
        agg_flat = _edge_kernel()(
            xl2.reshape(2 * N, HALF), wef_all[l].reshape(2 * E, HALF),
            src.reshape(NCHUNK, CB), dst.reshape(NCHUNK, CB), zer)
        agg2 = agg_flat.reshape(2, N, HALF)
        if l < 2:
            si, xl2 = _node_mid_kernel()(si, agg2, wl2s[l],
                                         wsis[l + 1], wl1s[l + 1])
    return _final_kernel()(si, agg2, wl2s[2], batch3)


# speedup vs baseline: 1.0660x; 1.0023x over previous
import functools
import math

import jax
import jax.numpy as jnp
import numpy as np
from jax import lax
from jax.experimental import pallas as pl
from jax.experimental.pallas import tpu as pltpu
from jax.experimental.pallas import tpu_sc as plsc

MAX_RADIUS = 2.0
N_BASIS = 10
N = 10000
E = 320000
D = 128
HID = 100
NG = 16

NC = 2
NS = 16
NW = NC * NS
LANES = 16

HALF = D // 2
ROWS_PER_TILE = N // NS
EC_LEN = E // NW
CB = 128
NCHUNK = E // CB
CHUNK_ITERS = -(-NCHUNK // NS)

_f32 = jnp.float32


def _sc_mesh():
    return plsc.VectorSubcoreMesh(
        core_axis_name="c", subcore_axis_name="s", num_cores=NC, num_subcores=NS
    )


def _len2_body(px_hbm, py_hbm, pz_hbm, src_hbm, dst_hbm, out_hbm,
               px_v, py_v, pz_v, src_v, dst_v, out_v):
    c = lax.axis_index("c")
    s = lax.axis_index("s")
    wid = s * NC + c
    base = wid * EC_LEN
    pltpu.sync_copy(px_hbm, px_v)
    pltpu.sync_copy(py_hbm, py_v)
    pltpu.sync_copy(pz_hbm, pz_v)
    pltpu.sync_copy(src_hbm.at[pl.ds(base, EC_LEN)], src_v)
    pltpu.sync_copy(dst_hbm.at[pl.ds(base, EC_LEN)], dst_v)

    def body(i, carry):
        si = src_v[pl.ds(i * LANES, LANES)]
        di = dst_v[pl.ds(i * LANES, LANES)]
        ax = plsc.load_gather(px_v, [si]) - plsc.load_gather(px_v, [di])
        ay = plsc.load_gather(py_v, [si]) - plsc.load_gather(py_v, [di])
        az = plsc.load_gather(pz_v, [si]) - plsc.load_gather(pz_v, [di])
        out_v[pl.ds(i * LANES, LANES)] = ax * ax + ay * ay + az * az
        return carry

    lax.fori_loop(0, EC_LEN // LANES, body, 0)
    pltpu.sync_copy(out_v, out_hbm.at[pl.ds(base, EC_LEN)])


@functools.cache
def _len2_kernel():
    return pl.kernel(
        _len2_body,
        out_type=jax.ShapeDtypeStruct((E,), _f32),
        mesh=_sc_mesh(),
        compiler_params=pltpu.CompilerParams(needs_layout_passes=False, use_tc_tiling_on_sc=False),
        scratch_types=[
            pltpu.VMEM((N,), _f32),
            pltpu.VMEM((N,), _f32),
            pltpu.VMEM((N,), _f32),
            pltpu.VMEM((EC_LEN,), jnp.int32),
            pltpu.VMEM((EC_LEN,), jnp.int32),
            pltpu.VMEM((EC_LEN,), _f32),
        ],
    )


G = 2
TPC = -(-(-(-NCHUNK // NS)) // G) * G
NGROUP = TPC // G


def _edge_body(xl_hbm, wef_hbm, src2_hbm, dst2_hbm, zer_hbm, agg_hbm,
               xl_sh, agg_sh,
               srcA, dstA, srcB, dstB,
               wefA, wefB, gx0, gx1,
               sem_iA, sem_iB, sem_wA, sem_wB, gsem, ssem):
    c = lax.axis_index("c")
    s = lax.axis_index("s")
    rbase = s * ROWS_PER_TILE
    pltpu.sync_copy(xl_hbm.at[pl.ds(c * N + rbase, ROWS_PER_TILE)],
                    xl_sh.at[pl.ds(rbase, ROWS_PER_TILE)])
    pltpu.sync_copy(zer_hbm.at[pl.ds(rbase, ROWS_PER_TILE)],
                    agg_sh.at[pl.ds(rbase, ROWS_PER_TILE)])
    plsc.subcore_barrier()

    IDX = ((srcA, dstA, sem_iA), (srcB, dstB, sem_iB))
    WEFS = (wefA, wefB)
    SW = (sem_wA, sem_wB)
    GXS = (gx0, gx1)

    tbase = s * TPC
    tend = jnp.minimum(NCHUNK, tbase + TPC)

    def issue_group(g, p):
        gb = jnp.minimum(tbase + g * G, NCHUNK - G)
        srcb, dstb, semi = IDX[p]
        pltpu.async_copy(src2_hbm.at[pl.ds(gb, G)], srcb, semi)
        pltpu.async_copy(dst2_hbm.at[pl.ds(gb, G)], dstb, semi)
        pltpu.async_copy(wef_hbm.at[pl.ds(c * E + gb * CB, G * CB)],
                         WEFS[p], SW[p])

    def wait_idx(p):
        srcb, dstb, semi = IDX[p]
        pltpu.make_async_copy(src2_hbm.at[pl.ds(0, G)], srcb, semi).wait()
        pltpu.make_async_copy(dst2_hbm.at[pl.ds(0, G)], dstb, semi).wait()

    def wait_wef(p):
        pltpu.make_async_copy(wef_hbm.at[pl.ds(c * E, G * CB)],
                              WEFS[p], SW[p]).wait()

    def group_step(g, p):
        q = 1 - p
        srcb, dstb, _ = IDX[p]
        srcq, dstq, _ = IDX[q]
        gbase = tbase + g * G
        wait_idx(p)
        for j in range(G):
            kp = gbase - G + j

            @pl.when(jnp.logical_and(kp >= tbase, kp < tend))
            def _():
                pltpu.make_async_copy(GXS[j], agg_sh.at[dstq.at[j]],
                                      ssem).wait()
        for j in range(G):
            pltpu.async_copy(xl_sh.at[srcb.at[j]], GXS[j], gsem)
        issue_group(g + 1, q)
        wait_wef(p)
        for j in range(G):
            pltpu.make_async_copy(xl_sh.at[srcb.at[j]], GXS[j], gsem).wait()
        wefv = WEFS[p]
        for j in range(G):
            gxv = GXS[j]

            def mul(b4, carry):
                for bb in range(4):
                    b = b4 * 4 + bb
                    for m in range(HALF // LANES):
                        slc = pl.ds(m * LANES, LANES)
                        gxv[b, slc] = gxv[b, slc] * wefv[j * CB + b, slc]
                return carry

            lax.fori_loop(0, CB // 4, mul, 0)
            kj = gbase + j

            @pl.when(kj < tend)
            def _():
                pltpu.async_copy(gxv, agg_sh.at[dstb.at[j]], ssem, add=True)

    issue_group(0, 0)

    def two(t2, carry):
        group_step(t2 * 2, 0)
        group_step(t2 * 2 + 1, 1)
        return carry

    lax.fori_loop(0, NGROUP // 2, two, 0)
    if NGROUP % 2:
        group_step(NGROUP - 1, 0)

    srcq, dstq, _ = IDX[(NGROUP - 1) % 2]
    for j in range(G):
        kp = tbase + (NGROUP - 1) * G + j

        @pl.when(kp < tend)
        def _():
            pltpu.make_async_copy(GXS[j], agg_sh.at[dstq.at[j]], ssem).wait()
    wait_idx(NGROUP % 2)
    wait_wef(NGROUP % 2)

    plsc.subcore_barrier()
    pltpu.sync_copy(agg_sh.at[pl.ds(rbase, ROWS_PER_TILE)],
                    agg_hbm.at[pl.ds(c * N + rbase, ROWS_PER_TILE)])


@functools.cache
def _edge_kernel():
    return pl.kernel(
        _edge_body,
        out_type=jax.ShapeDtypeStruct((2 * N, HALF), _f32),
        mesh=_sc_mesh(),
        compiler_params=pltpu.CompilerParams(needs_layout_passes=False, use_tc_tiling_on_sc=False),
        scratch_types=[
            pltpu.VMEM_SHARED((N, HALF), _f32),
            pltpu.VMEM_SHARED((N, HALF), _f32),
            pltpu.VMEM((G, CB), jnp.int32),
            pltpu.VMEM((G, CB), jnp.int32),
            pltpu.VMEM((G, CB), jnp.int32),
            pltpu.VMEM((G, CB), jnp.int32),
            pltpu.VMEM((G * CB, HALF), _f32),
            pltpu.VMEM((G * CB, HALF), _f32),
            pltpu.VMEM((CB, HALF), _f32),
            pltpu.VMEM((CB, HALF), _f32),
        ] + [pltpu.SemaphoreType.DMA] * 6,
    )


EB = 2560
SQRT2 = math.sqrt(2.0)


def _wef_body(len2_ref, w1a_ref, w2a_ref, w1b_ref, w2b_ref, w1c_ref, w2c_ref,
              outa_ref, outb_ref, outc_ref):
    l2 = len2_ref[0, 0, :]
    length = jnp.sqrt(l2 + 1e-12)
    centers = lax.broadcasted_iota(jnp.int32, (1, N_BASIS), 1).astype(_f32) * (
        MAX_RADIUS / (N_BASIS - 1))
    inv_sigma = (N_BASIS - 1) / MAX_RADIUS
    diff = (length[:, None] - centers) * inv_sigma
    emb = jnp.exp(-diff * diff)
    u = 2.0 * (length * (1.0 / MAX_RADIUS) - 1.0)
    y = (1.0 - jnp.cos(jnp.pi * u)) * 0.5
    y = jnp.where(u > 0.0, 0.0, y)
    y = jnp.where(u < -1.0, 1.0, y)
    for w1_ref, w2_ref, out_ref in ((w1a_ref, w2a_ref, outa_ref),
                                    (w1b_ref, w2b_ref, outb_ref),
                                    (w1c_ref, w2c_ref, outc_ref)):
        h1 = jnp.maximum(jnp.dot(emb, w1_ref[...],
                                 preferred_element_type=_f32), 0.0) * SQRT2
        w = jnp.dot(h1, w2_ref[...],
                    preferred_element_type=_f32) * (1.0 / math.sqrt(HID))
        wef = w * y[:, None]
        out_ref[0] = wef[:, :HALF]
        out_ref[1] = wef[:, HALF:]


@functools.cache
def _wef_kernel():
    wspec = [
        pl.BlockSpec((N_BASIS, HID), lambda i: (0, 0)),
        pl.BlockSpec((HID, D), lambda i: (0, 0)),
    ]
    ospec = pl.BlockSpec((2, EB, HALF), lambda i: (0, i, 0))
    oshape = jax.ShapeDtypeStruct((2, E, HALF), _f32)
    return pl.pallas_call(
        _wef_body,
        grid=(E // EB,),
        in_specs=[pl.BlockSpec((1, 1, EB), lambda i: (i, 0, 0))] + wspec * 3,
        out_specs=[ospec] * 3,
        out_shape=[oshape] * 3,
    )


RB = 2000
INV_SQRT_D = 1.0 / math.sqrt(D)
AGG_SCALE = 0.5 / math.sqrt(32.0 * D)


def _node0_body(x_ref, wsi_ref, wl1_ref, si_ref, xl_ref):
    h = x_ref[...]
    si_ref[...] = jnp.dot(h, wsi_ref[...], preferred_element_type=_f32) * INV_SQRT_D
    xl = jnp.dot(h, wl1_ref[...], preferred_element_type=_f32) * INV_SQRT_D
    xl_ref[0] = xl[:, :HALF]
    xl_ref[1] = xl[:, HALF:]


def _node_mid_body(sip_ref, agg_ref, wl2_ref, wsi_ref, wl1_ref, si_ref, xl_ref):
    agg = jnp.concatenate([agg_ref[0], agg_ref[1]], axis=-1)
    h = sip_ref[...] + jnp.dot(agg, wl2_ref[...],
                               preferred_element_type=_f32) * AGG_SCALE
    h = jnp.maximum(h, 0.0)
    si_ref[...] = jnp.dot(h, wsi_ref[...], preferred_element_type=_f32) * INV_SQRT_D
    xl = jnp.dot(h, wl1_ref[...], preferred_element_type=_f32) * INV_SQRT_D
    xl_ref[0] = xl[:, :HALF]
    xl_ref[1] = xl[:, HALF:]


@functools.cache
def _node0_kernel():
    return pl.pallas_call(
        _node0_body,
        grid=(N // RB,),
        in_specs=[
            pl.BlockSpec((RB, D), lambda i: (i, 0)),
            pl.BlockSpec((D, D), lambda i: (0, 0)),
            pl.BlockSpec((D, D), lambda i: (0, 0)),
        ],
        out_specs=[
            pl.BlockSpec((RB, D), lambda i: (i, 0)),
            pl.BlockSpec((2, RB, HALF), lambda i: (0, i, 0)),
        ],
        out_shape=[
            jax.ShapeDtypeStruct((N, D), _f32),
            jax.ShapeDtypeStruct((2, N, HALF), _f32),
        ],
    )


@functools.cache
def _node_mid_kernel():
    return pl.pallas_call(
        _node_mid_body,
        grid=(N // RB,),
        in_specs=[
            pl.BlockSpec((RB, D), lambda i: (i, 0)),
            pl.BlockSpec((2, RB, HALF), lambda i: (0, i, 0)),
            pl.BlockSpec((D, D), lambda i: (0, 0)),
            pl.BlockSpec((D, D), lambda i: (0, 0)),
            pl.BlockSpec((D, D), lambda i: (0, 0)),
        ],
        out_specs=[
            pl.BlockSpec((RB, D), lambda i: (i, 0)),
            pl.BlockSpec((2, RB, HALF), lambda i: (0, i, 0)),
        ],
        out_shape=[
            jax.ShapeDtypeStruct((N, D), _f32),
            jax.ShapeDtypeStruct((2, N, HALF), _f32),
        ],
    )


INV_SQRT_NODES = 1.0 / math.sqrt(625.0)


def _final_body(sip_ref, agg_ref, wl2_ref, batch_ref, out_ref):
    agg = jnp.concatenate([agg_ref[0], agg_ref[1]], axis=-1)
    h = sip_ref[...] + jnp.dot(agg, wl2_ref[...],
                               preferred_element_type=_f32) * AGG_SCALE
    b = batch_ref[0, 0, :]
    gids = lax.broadcasted_iota(jnp.int32, (NG, RB), 0).astype(_f32)
    m = jnp.where(jnp.equal(b[None, :], gids), 1.0, 0.0)
    contrib = jnp.dot(m, h, preferred_element_type=_f32) * INV_SQRT_NODES

    @pl.when(pl.program_id(0) == 0)
    def _():
        out_ref[...] = jnp.zeros_like(out_ref)

    out_ref[...] += contrib


@functools.cache
def _final_kernel():
    return pl.pallas_call(
        _final_body,
        grid=(N // RB,),
        in_specs=[
            pl.BlockSpec((RB, D), lambda i: (i, 0)),
            pl.BlockSpec((2, RB, HALF), lambda i: (0, i, 0)),
            pl.BlockSpec((D, D), lambda i: (0, 0)),
            pl.BlockSpec((1, 1, RB), lambda i: (i, 0, 0)),
        ],
        out_specs=pl.BlockSpec((NG, D), lambda i: (0, 0)),
        out_shape=jax.ShapeDtypeStruct((NG, D), _f32),
    )


def kernel(pos, x, z, batch, edge_src, edge_dst,
           Wsi0, Wl1_0, Wfc1_0, Wfc2_0, Wl2_0,
           Wsi1, Wl1_1, Wfc1_1, Wfc2_1, Wl2_1,
           Wsi2, Wl1_2, Wfc1_2, Wfc2_2, Wl2_2):
    del z
    px = jnp.asarray(pos[:, 0], _f32)
    py = jnp.asarray(pos[:, 1], _f32)
    pz = jnp.asarray(pos[:, 2], _f32)
    src = edge_src.astype(jnp.int32)
    dst = edge_dst.astype(jnp.int32)

    len2 = _len2_kernel()(px, py, pz, src, dst)
    len2_3d = len2.reshape(E // EB, 1, EB)
    zer = jnp.zeros((N, HALF), _f32)
    batch3 = batch.astype(_f32).reshape(N // RB, 1, RB)

    wsis = [Wsi0[:, 0, :], Wsi1[:, 0, :], Wsi2[:, 0, :]]
    wl1s = [Wl1_0[:, 0, :], Wl1_1[:, 0, :], Wl1_2[:, 0, :]]
    wl2s = [Wl2_0[:, 0, :], Wl2_1[:, 0, :], Wl2_2[:, 0, :]]
    wfc1s = [Wfc1_0, Wfc1_1, Wfc1_2]
    wfc2s = [Wfc2_0, Wfc2_1, Wfc2_2]

    si, xl2 = _node0_kernel()(x, wsis[0], wl1s[0])
    wef_all = _wef_kernel()(len2_3d, wfc1s[0], wfc2s[0], wfc1s[1], wfc2s[1],
                            wfc1s[2], wfc2s[2])
    for l in range(3):
        agg_flat = _edge_kernel()(
            xl2.reshape(2 * N, HALF), wef_all[l].reshape(2 * E, HALF),
            src.reshape(NCHUNK, CB), dst.reshape(NCHUNK, CB), zer)
        agg2 = agg_flat.reshape(2, N, HALF)
        if l < 2:
            si, xl2 = _node_mid_kernel()(si, agg2, wl2s[l],
                                         wsis[l + 1], wl1s[l + 1])
    return _final_kernel()(si, agg2, wl2s[2], batch3)
